# baseline (speedup 1.0000x reference)
import jax
import jax.numpy as jnp
from jax.experimental import pallas as pl
from jax.experimental.pallas import tpu as pltpu

HS = 501
HSP = 512
N = 8
NVT = 10
NVTP = 16
NZ = 56


def _pad_to(x, shape):
    return jnp.pad(x, [(0, s - d) for s, d in zip(shape, x.shape)])


def _body(nt_ref, et_ref, am_ref, wih_ref, bih_ref, whh_ref, bhh_ref,
          wg_ref, wm_ref, bg_ref, w12_ref, b12_ref, out_ref,
          h_scr, wih_s, whh_s, wgm_s):
    f32 = jnp.float32
    bf16 = jnp.bfloat16
    BB = nt_ref.shape[0]

    wih_s[...] = jnp.zeros((3 * HSP, NVTP), bf16)
    whh_s[...] = jnp.zeros((3 * HSP, HSP), bf16)
    wgm_s[...] = jnp.zeros((2 * HSP, HSP), bf16)
    for g in range(3):
        wih_s[g * HSP:g * HSP + HS, :NVT] = wih_ref[g * HS:(g + 1) * HS, :].astype(bf16)
        whh_s[g * HSP:g * HSP + HS, :HS] = whh_ref[g * HS:(g + 1) * HS, :].astype(bf16)
    wgm_s[0:HS, :HS + N] = wg_ref[...].astype(bf16)
    wgm_s[HSP:HSP + HS, :HS + N] = wm_ref[...].astype(bf16)

    def dot(a, b):
        return jax.lax.dot_general(a.astype(bf16), b,
                                   (((1,), (1,)), ((), ())),
                                   preferred_element_type=f32)

    lane16 = jax.lax.broadcasted_iota(jnp.int32, (BB, NVTP), 1)
    lane512 = jax.lax.broadcasted_iota(jnp.int32, (1, HSP), 1)

    for v in range(N):
        xv = (lane16 == nt_ref[:, v:v + 1]).astype(f32)
        gi = dot(xv, wih_s[...]) + bih_ref[0:1, :]
        if v == 0:
            hin = jnp.zeros((BB, HSP), f32)
            gh = jnp.zeros((BB, 3 * HSP), f32) + bhh_ref[0:1, :]
        else:
            hin = jnp.zeros((BB, HSP), f32)
            for x in range(v):
                ws = et_ref[:, x, v:v + 1].astype(f32) * 10.0 + 1.0
                m = am_ref[:, x, v:v + 1].astype(f32)
                vid_row = (lane512 == HS + x).astype(f32)
                msg = m * (ws * h_scr[x] + vid_row)
                gm = dot(msg, wgm_s[...])
                gate = jax.nn.sigmoid(gm[:, :HSP] + bg_ref[0:1, :])
                hin = hin + gate * gm[:, HSP:]
            gh = dot(hin, whh_s[...]) + bhh_ref[0:1, :]
        r = jax.nn.sigmoid(gi[:, :HSP] + gh[:, :HSP])
        z = jax.nn.sigmoid(gi[:, HSP:2 * HSP] + gh[:, HSP:2 * HSP])
        n = jnp.tanh(gi[:, 2 * HSP:] + r * gh[:, 2 * HSP:])
        hv = (1.0 - z) * n + z * hin
        if v < N - 1:
            h_scr[v] = hv
        else:
            out_ref[...] = dot(hv, w12_ref[...]) + b12_ref[0:1, :]


def kernel(node_types, edge_type, adj_mask, W_ih, W_hh, b_ih, b_hh,
           Wg, bg, Wm, W1, b1, W2, b2):
    f32 = jnp.float32
    bf16 = jnp.bfloat16
    B = node_types.shape[0]

    def bias3(b):
        return jnp.concatenate(
            [_pad_to(b[i * HS:(i + 1) * HS], (HSP,)) for i in range(3)])[None]

    bih = bias3(b_ih)
    bhh = bias3(b_hh)
    bgp = _pad_to(bg, (HSP,))[None]
    w12 = _pad_to(jnp.concatenate([W1, W2]), (2 * NZ, HSP)).astype(bf16)
    b12 = jnp.concatenate([b1, b2])[None]

    nt = node_types.astype(jnp.int32)
    et = edge_type.astype(jnp.int32)
    am = adj_mask.astype(jnp.int32)

    BB = 256
    nblk = B // BB
    const = lambda i: (0, 0)
    const3 = lambda i: (0, 0, 0)
    blk = lambda i: (i, 0)
    out = pl.pallas_call(
        _body,
        grid=(nblk,),
        in_specs=[
            pl.BlockSpec((BB, N), blk),
            pl.BlockSpec((BB, N, N), lambda i: (i, 0, 0)),
            pl.BlockSpec((BB, N, N), lambda i: (i, 0, 0)),
            pl.BlockSpec((3 * HS, NVT), const),
            pl.BlockSpec((1, 3 * HSP), const),
            pl.BlockSpec((3 * HS, HS), const),
            pl.BlockSpec((1, 3 * HSP), const),
            pl.BlockSpec((HS, HS + N), const),
            pl.BlockSpec((HS, HS + N), const),
            pl.BlockSpec((1, HSP), const),
            pl.BlockSpec((2 * NZ, HSP), const),
            pl.BlockSpec((1, 2 * NZ), const),
        ],
        out_specs=pl.BlockSpec((BB, 2 * NZ), blk),
        out_shape=jax.ShapeDtypeStruct((B, 2 * NZ), f32),
        scratch_shapes=[pltpu.VMEM((N - 1, BB, HSP), f32),
                        pltpu.VMEM((3 * HSP, NVTP), bf16),
                        pltpu.VMEM((3 * HSP, HSP), bf16),
                        pltpu.VMEM((2 * HSP, HSP), bf16)],
    )(nt, et, am, W_ih, bih, W_hh, bhh, Wg, Wm, bgp, w12, b12)
    return out[:, :NZ], out[:, NZ:]


# speedup vs baseline: 1.5866x; 1.5866x over previous
import jax
import jax.numpy as jnp
from jax.experimental import pallas as pl
from jax.experimental.pallas import tpu as pltpu

HS = 501
HSP = 512
N = 8
NVT = 10
NVTP = 16
NZ = 56


def _pad_to(x, shape):
    return jnp.pad(x, [(0, s - d) for s, d in zip(shape, x.shape)])


def _body(x_ref, w_ref, m_ref, wih_ref, bih_ref, whh_ref, bhh_ref,
          wg_ref, wm_ref, bg_ref, vid_ref, w12_ref, b12_ref, out_ref,
          h_scr, wih_s, whh_s, wgm_s):
    f32 = jnp.float32
    bf16 = jnp.bfloat16

    wih_s[...] = jnp.zeros(wih_s.shape, bf16)
    whh_s[...] = jnp.zeros(whh_s.shape, bf16)
    wgm_s[...] = jnp.zeros(wgm_s.shape, bf16)
    for g in range(3):
        wih_s[g * HSP:g * HSP + HS, :NVT] = wih_ref[g].astype(bf16)
        whh_s[g * HSP:g * HSP + HS, :HS] = whh_ref[g].astype(bf16)
    wgm_s[0:HS, :HS + N] = wg_ref[...].astype(bf16)
    wgm_s[HSP:HSP + HS, :HS + N] = wm_ref[...].astype(bf16)

    def dot(a, b):
        return jax.lax.dot_general(a.astype(jnp.bfloat16), b,
                                   (((1,), (1,)), ((), ())),
                                   preferred_element_type=f32)

    BB = x_ref.shape[0]
    for v in range(N):
        xv = x_ref[:, v * NVTP:(v + 1) * NVTP]
        gi = dot(xv, wih_s[...]) + bih_ref[0:1, :]
        if v == 0:
            hin = jnp.zeros((BB, HSP), f32)
            gh = jnp.zeros((BB, 3 * HSP), f32) + bhh_ref[0:1, :]
        else:
            hin = jnp.zeros((BB, HSP), f32)
            for x in range(v):
                idx = x * N + v
                ws = w_ref[:, idx:idx + 1]
                m = m_ref[:, idx:idx + 1]
                msg = m * (ws * h_scr[x] + vid_ref[x:x + 1, :])
                gm = dot(msg, wgm_s[...])
                gate = jax.nn.sigmoid(gm[:, :HSP] + bg_ref[0:1, :])
                hin = hin + gate * gm[:, HSP:]
            gh = dot(hin, whh_s[...]) + bhh_ref[0:1, :]
        r = jax.nn.sigmoid(gi[:, :HSP] + gh[:, :HSP])
        z = jax.nn.sigmoid(gi[:, HSP:2 * HSP] + gh[:, HSP:2 * HSP])
        n = jnp.tanh(gi[:, 2 * HSP:] + r * gh[:, 2 * HSP:])
        hv = (1.0 - z) * n + z * hin
        if v < N - 1:
            h_scr[v] = hv
        else:
            out_ref[...] = dot(hv, w12_ref[...]) + b12_ref[0:1, :]


def kernel(node_types, edge_type, adj_mask, W_ih, W_hh, b_ih, b_hh,
           Wg, bg, Wm, W1, b1, W2, b2):
    f32 = jnp.float32
    bf16 = jnp.bfloat16
    B = node_types.shape[0]

    wih3 = W_ih.reshape(3, HS, NVT)
    whh3 = W_hh.reshape(3, HS, HS)

    def bias3(b):
        return jnp.concatenate(
            [_pad_to(b[i * HS:(i + 1) * HS], (HSP,)) for i in range(3)])[None]

    bih = bias3(b_ih)
    bhh = bias3(b_hh)

    bgp = _pad_to(bg, (HSP,))[None]
    vid = _pad_to(jnp.concatenate(
        [jnp.zeros((N, HS), f32), jnp.eye(N, dtype=f32)], axis=1),
        (N, HSP))

    w12 = _pad_to(jnp.concatenate([W1, W2]), (2 * NZ, HSP)).astype(bf16)
    b12 = jnp.concatenate([b1, b2])[None]

    X = jax.nn.one_hot(node_types, NVT, dtype=f32)
    X = _pad_to(X, (B, N, NVTP)).reshape(B, N * NVTP)
    dag = jnp.triu(jnp.ones((N, N), f32), k=1)[None]
    msk = (adj_mask.astype(f32) * dag).reshape(B, N * N)
    wsc = (edge_type.astype(f32) * 10.0 + 1.0).reshape(B, N * N)

    BB = 256
    nblk = B // BB
    const = lambda i: (0, 0)
    blk = lambda i: (i, 0)
    out = pl.pallas_call(
        _body,
        grid=(nblk,),
        in_specs=[
            pl.BlockSpec((BB, N * NVTP), blk),
            pl.BlockSpec((BB, N * N), blk),
            pl.BlockSpec((BB, N * N), blk),
            pl.BlockSpec((3, HS, NVT), lambda i: (0, 0, 0)),
            pl.BlockSpec((1, 3 * HSP), const),
            pl.BlockSpec((3, HS, HS), lambda i: (0, 0, 0)),
            pl.BlockSpec((1, 3 * HSP), const),
            pl.BlockSpec((HS, HS + N), const),
            pl.BlockSpec((HS, HS + N), const),
            pl.BlockSpec((1, HSP), const),
            pl.BlockSpec((N, HSP), const),
            pl.BlockSpec((2 * NZ, HSP), const),
            pl.BlockSpec((1, 2 * NZ), const),
        ],
        out_specs=pl.BlockSpec((BB, 2 * NZ), blk),
        out_shape=jax.ShapeDtypeStruct((B, 2 * NZ), f32),
        scratch_shapes=[pltpu.VMEM((N - 1, BB, HSP), f32),
                        pltpu.VMEM((3 * HSP, NVTP), bf16),
                        pltpu.VMEM((3 * HSP, HSP), bf16),
                        pltpu.VMEM((2 * HSP, HSP), bf16)],
    )(X, wsc, msk, wih3, bih, whh3, bhh, Wg, Wm, bgp, vid, w12, b12)
    return out[:, :NZ], out[:, NZ:]


# speedup vs baseline: 1.8209x; 1.1477x over previous
import jax
import jax.numpy as jnp
from jax.experimental import pallas as pl
from jax.experimental.pallas import tpu as pltpu

HS = 501
HSP = 512
N = 8
NVT = 10
NVTP = 16
NZ = 56


def _pad_to(x, shape):
    return jnp.pad(x, [(0, s - d) for s, d in zip(shape, x.shape)])


def _body(x_ref, w_ref, m_ref, wih_ref, bih_ref, whh_ref, bhh_ref,
          wgm_ref, bg_ref, vid_ref, w12_ref, b12_ref, out_ref,
          h_scr, msg_scr):
    f32 = jnp.float32
    bf16 = jnp.bfloat16

    def dot(a, b):
        return jax.lax.dot_general(a.astype(bf16), b,
                                   (((1,), (1,)), ((), ())),
                                   preferred_element_type=f32)

    BB = x_ref.shape[0]
    for v in range(N):
        xv = x_ref[:, v * NVTP:(v + 1) * NVTP]
        gi = dot(xv, wih_ref[...]) + bih_ref[0:1, :]
        if v == 0:
            hin = jnp.zeros((BB, HSP), f32)
            gh = jnp.zeros((BB, 3 * HSP), f32) + bhh_ref[0:1, :]
        else:
            for x in range(v):
                idx = x * N + v
                ws = w_ref[:, idx:idx + 1]
                m = m_ref[:, idx:idx + 1]
                msg_scr[pl.ds(x * BB, BB), :] = (
                    m * (ws * h_scr[x] + vid_ref[x:x + 1, :])).astype(bf16)
            gm = dot(msg_scr[pl.ds(0, v * BB), :], wgm_ref[...])
            gate = jax.nn.sigmoid(gm[:, :HSP] + bg_ref[0:1, :])
            gated = gate * gm[:, HSP:]
            hin = jnp.zeros((BB, HSP), f32)
            for x in range(v):
                hin = hin + gated[x * BB:(x + 1) * BB, :]
            gh = dot(hin, whh_ref[...]) + bhh_ref[0:1, :]
        r = jax.nn.sigmoid(gi[:, :HSP] + gh[:, :HSP])
        z = jax.nn.sigmoid(gi[:, HSP:2 * HSP] + gh[:, HSP:2 * HSP])
        n = jnp.tanh(gi[:, 2 * HSP:] + r * gh[:, 2 * HSP:])
        hv = (1.0 - z) * n + z * hin
        if v < N - 1:
            h_scr[v] = hv
        else:
            out_ref[...] = dot(hv, w12_ref[...]) + b12_ref[0:1, :]


def kernel(node_types, edge_type, adj_mask, W_ih, W_hh, b_ih, b_hh,
           Wg, bg, Wm, W1, b1, W2, b2):
    f32 = jnp.float32
    bf16 = jnp.bfloat16
    B = node_types.shape[0]

    def split3(W, kpad):
        return jnp.concatenate(
            [_pad_to(W[i * HS:(i + 1) * HS], (HSP, kpad)) for i in range(3)])

    wih = split3(W_ih, NVTP).astype(bf16)
    whh = split3(W_hh, HSP).astype(bf16)

    def bias3(b):
        return jnp.concatenate(
            [_pad_to(b[i * HS:(i + 1) * HS], (HSP,)) for i in range(3)])[None]

    bih = bias3(b_ih)
    bhh = bias3(b_hh)

    wgm = jnp.concatenate([_pad_to(Wg, (HSP, HSP)),
                           _pad_to(Wm, (HSP, HSP))]).astype(bf16)
    bgp = _pad_to(bg, (HSP,))[None]
    vid = _pad_to(jnp.concatenate(
        [jnp.zeros((N, HS), f32), jnp.eye(N, dtype=f32)], axis=1),
        (N, HSP))

    w12 = _pad_to(jnp.concatenate([W1, W2]), (2 * NZ, HSP)).astype(bf16)
    b12 = jnp.concatenate([b1, b2])[None]

    X = jax.nn.one_hot(node_types, NVT, dtype=f32)
    X = _pad_to(X, (B, N, NVTP)).reshape(B, N * NVTP)
    dag = jnp.triu(jnp.ones((N, N), f32), k=1)[None]
    msk = (adj_mask.astype(f32) * dag).reshape(B, N * N)
    wsc = (edge_type.astype(f32) * 10.0 + 1.0).reshape(B, N * N)

    BB = 256
    nblk = B // BB
    const = lambda i: (0, 0)
    blk = lambda i: (i, 0)
    out = pl.pallas_call(
        _body,
        grid=(nblk,),
        in_specs=[
            pl.BlockSpec((BB, N * NVTP), blk),
            pl.BlockSpec((BB, N * N), blk),
            pl.BlockSpec((BB, N * N), blk),
            pl.BlockSpec((3 * HSP, NVTP), const),
            pl.BlockSpec((1, 3 * HSP), const),
            pl.BlockSpec((3 * HSP, HSP), const),
            pl.BlockSpec((1, 3 * HSP), const),
            pl.BlockSpec((2 * HSP, HSP), const),
            pl.BlockSpec((1, HSP), const),
            pl.BlockSpec((N, HSP), const),
            pl.BlockSpec((2 * NZ, HSP), const),
            pl.BlockSpec((1, 2 * NZ), const),
        ],
        out_specs=pl.BlockSpec((BB, 2 * NZ), blk),
        out_shape=jax.ShapeDtypeStruct((B, 2 * NZ), f32),
        scratch_shapes=[pltpu.VMEM((N - 1, BB, HSP), f32),
                        pltpu.VMEM(((N - 1) * BB, HSP), bf16)],
    )(X, wsc, msk, wih, bih, whh, bhh, wgm, bgp, vid, w12, b12)
    return out[:, :NZ], out[:, NZ:]


# speedup vs baseline: 1.8632x; 1.0232x over previous
import jax
import jax.numpy as jnp
from jax.experimental import pallas as pl
from jax.experimental.pallas import tpu as pltpu

HS = 501
HSP = 512
N = 8
NVT = 10
NVTP = 16
NZ = 56


def _pad_to(x, shape):
    return jnp.pad(x, [(0, s - d) for s, d in zip(shape, x.shape)])


def _body(x_ref, w_ref, m_ref, wih_ref, bih_ref, whh_ref, bhh_ref,
          wgm_ref, bg_ref, vid_ref, w12_ref, b12_ref, out_ref, h_scr):
    f32 = jnp.float32
    bf16 = jnp.bfloat16

    def dot(a, b):
        return jax.lax.dot_general(a.astype(bf16), b,
                                   (((1,), (1,)), ((), ())),
                                   preferred_element_type=f32)

    BB = x_ref.shape[0]
    for v in range(N):
        xv = x_ref[:, v * NVTP:(v + 1) * NVTP]
        gi = dot(xv, wih_ref[...]) + bih_ref[0:1, :]
        if v == 0:
            hin = jnp.zeros((BB, HSP), f32)
            gh = jnp.zeros((BB, 3 * HSP), f32) + bhh_ref[0:1, :]
        else:
            hin = jnp.zeros((BB, HSP), f32)
            for x in range(v):
                idx = x * N + v
                ws = w_ref[:, idx:idx + 1]
                m = m_ref[:, idx:idx + 1]
                msg = m * (ws * h_scr[x] + vid_ref[x:x + 1, :])
                gm = dot(msg, wgm_ref[...])
                gate = jax.nn.sigmoid(gm[:, :HSP] + bg_ref[0:1, :])
                hin = hin + gate * gm[:, HSP:]
            gh = None
        if v == 0:
            ghr = gh[:, :HSP]
            ghz = gh[:, HSP:2 * HSP]
            ghn = gh[:, 2 * HSP:]
        else:
            ghr = dot(hin, whh_ref[0:HSP, :]) + bhh_ref[0:1, 0:HSP]
            ghz = dot(hin, whh_ref[HSP:2 * HSP, :]) + bhh_ref[0:1, HSP:2 * HSP]
            ghn = dot(hin, whh_ref[2 * HSP:, :]) + bhh_ref[0:1, 2 * HSP:]
        r = jax.nn.sigmoid(gi[:, :HSP] + ghr)
        z = jax.nn.sigmoid(gi[:, HSP:2 * HSP] + ghz)
        n = jnp.tanh(gi[:, 2 * HSP:] + r * ghn)
        hv = (1.0 - z) * n + z * hin
        if v < N - 1:
            h_scr[v] = hv
        else:
            out_ref[...] = dot(hv, w12_ref[...]) + b12_ref[0:1, :]


def kernel(node_types, edge_type, adj_mask, W_ih, W_hh, b_ih, b_hh,
           Wg, bg, Wm, W1, b1, W2, b2):
    f32 = jnp.float32
    bf16 = jnp.bfloat16
    B = node_types.shape[0]

    def split3(W, kpad):
        return jnp.concatenate(
            [_pad_to(W[i * HS:(i + 1) * HS], (HSP, kpad)) for i in range(3)])

    wih = split3(W_ih, NVTP).astype(bf16)
    whh = split3(W_hh, HSP).astype(bf16)

    def bias3(b):
        return jnp.concatenate(
            [_pad_to(b[i * HS:(i + 1) * HS], (HSP,)) for i in range(3)])[None]

    bih = bias3(b_ih)
    bhh = bias3(b_hh)

    wgm = jnp.concatenate([_pad_to(Wg, (HSP, HSP)),
                           _pad_to(Wm, (HSP, HSP))]).astype(bf16)
    bgp = _pad_to(bg, (HSP,))[None]
    vid = _pad_to(jnp.concatenate(
        [jnp.zeros((N, HS), f32), jnp.eye(N, dtype=f32)], axis=1),
        (N, HSP))

    w12 = _pad_to(jnp.concatenate([W1, W2]), (2 * NZ, HSP)).astype(bf16)
    b12 = jnp.concatenate([b1, b2])[None]

    X = jax.nn.one_hot(node_types, NVT, dtype=f32)
    X = _pad_to(X, (B, N, NVTP)).reshape(B, N * NVTP)
    dag = jnp.triu(jnp.ones((N, N), f32), k=1)[None]
    msk = (adj_mask.astype(f32) * dag).reshape(B, N * N)
    wsc = (edge_type.astype(f32) * 10.0 + 1.0).reshape(B, N * N)

    BB = 256
    nblk = B // BB
    const = lambda i: (0, 0)
    blk = lambda i: (i, 0)
    out = pl.pallas_call(
        _body,
        grid=(nblk,),
        in_specs=[
            pl.BlockSpec((BB, N * NVTP), blk),
            pl.BlockSpec((BB, N * N), blk),
            pl.BlockSpec((BB, N * N), blk),
            pl.BlockSpec((3 * HSP, NVTP), const),
            pl.BlockSpec((1, 3 * HSP), const),
            pl.BlockSpec((3 * HSP, HSP), const),
            pl.BlockSpec((1, 3 * HSP), const),
            pl.BlockSpec((2 * HSP, HSP), const),
            pl.BlockSpec((1, HSP), const),
            pl.BlockSpec((N, HSP), const),
            pl.BlockSpec((2 * NZ, HSP), const),
            pl.BlockSpec((1, 2 * NZ), const),
        ],
        out_specs=pl.BlockSpec((BB, 2 * NZ), blk),
        out_shape=jax.ShapeDtypeStruct((B, 2 * NZ), f32),
        scratch_shapes=[pltpu.VMEM((N - 1, BB, HSP), f32)],
    )(X, wsc, msk, wih, bih, whh, bhh, wgm, bgp, vid, w12, b12)
    return out[:, :NZ], out[:, NZ:]
